# baseline (device time: 46310 ns/iter reference)
import jax
import jax.numpy as jnp
from jax import lax
from jax.experimental import pallas as pl
from jax.experimental.pallas import tpu as pltpu

N_DEV = 4


def _gelu(y):
    c = 0.7978845608028654
    return 0.5 * y * (1.0 + jnp.tanh(c * (y + 0.044715 * y * y * y)))


def kernel(x, w_mat):
    m, _ = x.shape
    _, n = w_mat.shape
    m_chunk = m // N_DEV

    def body(x_ref, w_ref, out_ref, comm_ref, send_sems, recv_sems):
        my = lax.axis_index("i")
        left = (my - 1) % N_DEV
        right = (my + 1) % N_DEV

        barrier = pltpu.get_barrier_semaphore()
        for nbr in (left, right):
            pl.semaphore_signal(
                barrier, inc=1,
                device_id=(nbr,), device_id_type=pl.DeviceIdType.MESH,
            )
        pl.semaphore_wait(barrier, 2)

        def partial_chunk(c):
            return jnp.dot(
                x_ref[pl.ds(c * m_chunk, m_chunk), :], w_ref[...],
                preferred_element_type=jnp.float32,
            )

        comm_ref[0] = partial_chunk((my - 1) % N_DEV)

        for h in range(N_DEV - 1):
            rdma = pltpu.make_async_remote_copy(
                src_ref=comm_ref.at[h],
                dst_ref=comm_ref.at[h + 1],
                send_sem=send_sems.at[h],
                recv_sem=recv_sems.at[h],
                device_id=(right,),
                device_id_type=pl.DeviceIdType.MESH,
            )
            rdma.start()
            p = partial_chunk((my - h - 2) % N_DEV)
            rdma.wait()
            if h < N_DEV - 2:
                comm_ref[h + 1] = comm_ref[h + 1] + p
            else:
                out_ref[...] = _gelu(comm_ref[h + 1] + p)

    return pl.pallas_call(
        body,
        out_shape=jax.ShapeDtypeStruct((m_chunk, n), jnp.float32),
        in_specs=[
            pl.BlockSpec(memory_space=pltpu.VMEM),
            pl.BlockSpec(memory_space=pltpu.VMEM),
        ],
        out_specs=pl.BlockSpec(memory_space=pltpu.VMEM),
        scratch_shapes=[
            pltpu.VMEM((N_DEV, m_chunk, n), jnp.float32),
            pltpu.SemaphoreType.DMA((N_DEV - 1,)),
            pltpu.SemaphoreType.DMA((N_DEV - 1,)),
        ],
        compiler_params=pltpu.CompilerParams(collective_id=0),
    )(x, w_mat)


# device time: 29349 ns/iter; 1.5779x vs baseline; 1.5779x over previous
import jax
import jax.numpy as jnp
from jax import lax
from jax.experimental import pallas as pl
from jax.experimental.pallas import tpu as pltpu

N_DEV = 4


def _gelu(y):
    c = 0.7978845608028654
    return 0.5 * y * (1.0 + jnp.tanh(c * (y + 0.044715 * y * y * y)))


def kernel(x, w_mat):
    m, _ = x.shape
    _, n = w_mat.shape
    m_chunk = m // N_DEV
    n_half = n // 2

    def body(x_ref, w_ref, out_ref, cw_ref, ccw_ref,
             cw_send, cw_recv, ccw_send, ccw_recv):
        my = lax.axis_index("i")
        left = (my - 1) % N_DEV
        right = (my + 1) % N_DEV

        barrier = pltpu.get_barrier_semaphore()
        for nbr in (left, right):
            pl.semaphore_signal(
                barrier, inc=1,
                device_id=(nbr,), device_id_type=pl.DeviceIdType.MESH,
            )
        pl.semaphore_wait(barrier, 2)

        def pchunk(c, lo, width):
            return jnp.dot(
                x_ref[pl.ds(c * m_chunk, m_chunk), :],
                w_ref[:, lo:lo + width],
                preferred_element_type=jnp.float32,
            )

        def hop(ring_ref, sends, recvs, h, dst):
            return pltpu.make_async_remote_copy(
                src_ref=ring_ref.at[h],
                dst_ref=ring_ref.at[h + 1],
                send_sem=sends.at[h],
                recv_sem=recvs.at[h],
                device_id=(dst,),
                device_id_type=pl.DeviceIdType.MESH,
            )

        cw_ref[0] = pchunk((my - 1) % N_DEV, 0, n_half)
        cw0 = hop(cw_ref, cw_send, cw_recv, 0, right)
        cw0.start()
        ccw_ref[0] = pchunk((my + 1) % N_DEV, n_half, n_half)
        ccw0 = hop(ccw_ref, ccw_send, ccw_recv, 0, left)
        ccw0.start()

        p2 = pchunk((my + 2) % N_DEV, 0, n)

        cw0.wait()
        cw_ref[1] = cw_ref[1] + p2[:, :n_half]
        cw1 = hop(cw_ref, cw_send, cw_recv, 1, right)
        cw1.start()
        ccw0.wait()
        ccw_ref[1] = ccw_ref[1] + p2[:, n_half:]
        ccw1 = hop(ccw_ref, ccw_send, ccw_recv, 1, left)
        ccw1.start()

        p_cw1 = pchunk((my + 1) % N_DEV, 0, n_half)
        p_ccw1 = pchunk((my - 1) % N_DEV, n_half, n_half)

        cw1.wait()
        cw_ref[2] = cw_ref[2] + p_cw1
        cw2 = hop(cw_ref, cw_send, cw_recv, 2, right)
        cw2.start()
        ccw1.wait()
        ccw_ref[2] = ccw_ref[2] + p_ccw1
        ccw2 = hop(ccw_ref, ccw_send, ccw_recv, 2, left)
        ccw2.start()

        p_own = pchunk(my, 0, n)
        cw2.wait()
        out_ref[:, :n_half] = _gelu(cw_ref[3] + p_own[:, :n_half])
        ccw2.wait()
        out_ref[:, n_half:] = _gelu(ccw_ref[3] + p_own[:, n_half:])

    return pl.pallas_call(
        body,
        out_shape=jax.ShapeDtypeStruct((m_chunk, n), jnp.float32),
        in_specs=[
            pl.BlockSpec(memory_space=pltpu.VMEM),
            pl.BlockSpec(memory_space=pltpu.VMEM),
        ],
        out_specs=pl.BlockSpec(memory_space=pltpu.VMEM),
        scratch_shapes=[
            pltpu.VMEM((N_DEV, m_chunk, n_half), jnp.float32),
            pltpu.VMEM((N_DEV, m_chunk, n_half), jnp.float32),
            pltpu.SemaphoreType.DMA((N_DEV - 1,)),
            pltpu.SemaphoreType.DMA((N_DEV - 1,)),
            pltpu.SemaphoreType.DMA((N_DEV - 1,)),
            pltpu.SemaphoreType.DMA((N_DEV - 1,)),
        ],
        compiler_params=pltpu.CompilerParams(collective_id=0),
    )(x, w_mat)


# device time: 25915 ns/iter; 1.7870x vs baseline; 1.1325x over previous
import jax
import jax.numpy as jnp
from jax import lax
from jax.experimental import pallas as pl
from jax.experimental.pallas import tpu as pltpu

N_DEV = 4
N_HOP = N_DEV - 1


def _gelu(y):
    c = 0.7978845608028654
    return 0.5 * y * (1.0 + jnp.tanh(c * (y + 0.044715 * y * y * y)))


def kernel(x, w_mat):
    m, _ = x.shape
    _, n = w_mat.shape
    m_chunk = m // N_DEV
    n_q = n // 4

    RINGS = (
        dict(lo=0, dirn=+1),
        dict(lo=2 * n_q, dirn=-1),
        dict(lo=1 * n_q, dirn=+1),
        dict(lo=3 * n_q, dirn=-1),
    )

    def body(x_ref, w_ref, out_ref, *scratch):
        ring_refs = scratch[0:4]
        send_sems = scratch[4:8]
        recv_sems = scratch[8:12]

        my = lax.axis_index("i")
        left = (my - 1) % N_DEV
        right = (my + 1) % N_DEV

        barrier = pltpu.get_barrier_semaphore()
        for nbr in (left, right):
            pl.semaphore_signal(
                barrier, inc=1,
                device_id=(nbr,), device_id_type=pl.DeviceIdType.MESH,
            )
        pl.semaphore_wait(barrier, 2)

        def pchunk(c, lo, width):
            return jnp.dot(
                x_ref[pl.ds(c * m_chunk, m_chunk), :],
                w_ref[:, lo:lo + width],
                preferred_element_type=jnp.float32,
            )

        def hop(k, h):
            r = RINGS[k]
            dst = right if r["dirn"] > 0 else left
            return pltpu.make_async_remote_copy(
                src_ref=ring_refs[k].at[h],
                dst_ref=ring_refs[k].at[h + 1],
                send_sem=send_sems[k].at[h],
                recv_sem=recv_sems[k].at[h],
                device_id=(dst,),
                device_id_type=pl.DeviceIdType.MESH,
            )

        rdmas = [[None] * N_HOP for _ in RINGS]
        for k, r in enumerate(RINGS):
            seed = (my - r["dirn"]) % N_DEV
            ring_refs[k][0] = pchunk(seed, r["lo"], n_q)
            rdmas[k][0] = hop(k, 0)
            rdmas[k][0].start()

        p = pchunk((my + 2) % N_DEV, 0, n)

        for h in range(N_HOP):
            for k, r in enumerate(RINGS):
                rdmas[k][h].wait()
                lo = r["lo"]
                acc = ring_refs[k][h + 1] + p[:, lo:lo + n_q]
                if h + 1 < N_HOP:
                    ring_refs[k][h + 1] = acc
                    rdmas[k][h + 1] = hop(k, h + 1)
                    rdmas[k][h + 1].start()
                else:
                    out_ref[:, lo:lo + n_q] = _gelu(acc)
            if h == 0:
                p = jnp.concatenate(
                    [
                        pchunk((my + 1) % N_DEV, 0, n // 2),
                        pchunk((my - 1) % N_DEV, n // 2, n // 2),
                    ],
                    axis=1,
                )
            elif h == 1:
                p = pchunk(my, 0, n)

    return pl.pallas_call(
        body,
        out_shape=jax.ShapeDtypeStruct((m_chunk, n), jnp.float32),
        in_specs=[
            pl.BlockSpec(memory_space=pltpu.VMEM),
            pl.BlockSpec(memory_space=pltpu.VMEM),
        ],
        out_specs=pl.BlockSpec(memory_space=pltpu.VMEM),
        scratch_shapes=(
            [pltpu.VMEM((N_DEV, m_chunk, n_q), jnp.float32)] * 4
            + [pltpu.SemaphoreType.DMA((N_HOP,))] * 8
        ),
        compiler_params=pltpu.CompilerParams(collective_id=0),
    )(x, w_mat)
